# baseline (speedup 1.0000x reference)
import jax
import jax.numpy as jnp
from jax.experimental import pallas as pl
from jax.experimental.pallas import tpu as pltpu

N_NODES = 8192
N_HYPER = 2048
IN_FT = 256
OUT_FT = 256

BLK_HH = 512
BLK_HG = 1024


def _phase1_body(hh_ref, x_ref, w1_ref, b1_ref, w2_ref, b2_ref,
                 he_ref, z_ref, h_scr):
    @pl.when(pl.program_id(0) == 0)
    def _():
        h32 = jax.lax.dot_general(
            x_ref[...].astype(jnp.bfloat16), w1_ref[...].astype(jnp.bfloat16),
            (((1,), (0,)), ((), ())), preferred_element_type=jnp.float32)
        h_scr[...] = (h32 + b1_ref[...]).astype(jnp.bfloat16)

    he32 = jax.lax.dot_general(
        hh_ref[...].astype(jnp.bfloat16), h_scr[...],
        (((1,), (0,)), ((), ())), preferred_element_type=jnp.float32)
    he32 = jnp.maximum(he32, 0.0)
    he_ref[...] = he32
    z32 = jax.lax.dot_general(
        he32.astype(jnp.bfloat16), w2_ref[...].astype(jnp.bfloat16),
        (((1,), (0,)), ((), ())), preferred_element_type=jnp.float32)
    z_ref[...] = (z32 + b2_ref[...]).astype(jnp.bfloat16)


def _phase2_body(hg_ref, z_ref, out_ref):
    o32 = jax.lax.dot_general(
        hg_ref[...].astype(jnp.bfloat16), z_ref[...],
        (((1,), (0,)), ((), ())), preferred_element_type=jnp.float32)
    out_ref[...] = jnp.maximum(o32, 0.0)


def kernel(x, norm_HH, norm_HG, weight1, bias1, weight2, bias2):
    b1 = bias1.reshape(1, OUT_FT)
    b2 = bias2.reshape(1, IN_FT)

    hyper_emb, z_bf = pl.pallas_call(
        _phase1_body,
        grid=(N_HYPER // BLK_HH,),
        in_specs=[
            pl.BlockSpec((BLK_HH, N_NODES), lambda i: (i, 0)),
            pl.BlockSpec((N_NODES, IN_FT), lambda i: (0, 0)),
            pl.BlockSpec((IN_FT, OUT_FT), lambda i: (0, 0)),
            pl.BlockSpec((1, OUT_FT), lambda i: (0, 0)),
            pl.BlockSpec((OUT_FT, IN_FT), lambda i: (0, 0)),
            pl.BlockSpec((1, IN_FT), lambda i: (0, 0)),
        ],
        out_specs=[
            pl.BlockSpec((BLK_HH, OUT_FT), lambda i: (i, 0)),
            pl.BlockSpec((BLK_HH, IN_FT), lambda i: (i, 0)),
        ],
        out_shape=[
            jax.ShapeDtypeStruct((N_HYPER, OUT_FT), jnp.float32),
            jax.ShapeDtypeStruct((N_HYPER, IN_FT), jnp.bfloat16),
        ],
        scratch_shapes=[pltpu.VMEM((N_NODES, OUT_FT), jnp.bfloat16)],
    )(norm_HH, x, weight1, b1, weight2, b2)

    out = pl.pallas_call(
        _phase2_body,
        grid=(N_NODES // BLK_HG,),
        in_specs=[
            pl.BlockSpec((BLK_HG, N_HYPER), lambda i: (i, 0)),
            pl.BlockSpec((N_HYPER, IN_FT), lambda i: (0, 0)),
        ],
        out_specs=pl.BlockSpec((BLK_HG, IN_FT), lambda i: (i, 0)),
        out_shape=jax.ShapeDtypeStruct((N_NODES, IN_FT), jnp.float32),
    )(norm_HG, z_bf)

    return (out, hyper_emb)
